# Initial kernel scaffold; baseline (speedup 1.0000x reference)
import jax
import jax.numpy as jnp
from jax.experimental import pallas as pl


def kernel(indices, table):
    raise NotImplementedError("write your pallas kernel here")


# speedup vs baseline: 1.5841x; 1.5841x over previous
import functools

import jax
import jax.numpy as jnp
from jax import lax
from jax.experimental import pallas as pl
from jax.experimental.pallas import tpu as pltpu
from jax.experimental.pallas import tpu_sc as plsc

_B = 4096
_F = 26
_D = 64
_R = _B * _F
_NC = 2
_NS = 16
_NW = _NC * _NS
_CHUNK = 128
_RPW = _R // _NW
_CPW = _RPW // _CHUNK

_mesh = plsc.VectorSubcoreMesh(core_axis_name="c", subcore_axis_name="s")


@functools.partial(
    pl.kernel,
    mesh=_mesh,
    compiler_params=pltpu.CompilerParams(use_tc_tiling_on_sc=False),
    out_type=jax.ShapeDtypeStruct((_R, _D), jnp.float32),
    scratch_types=[
        pltpu.VMEM((_RPW,), jnp.int32),
        pltpu.VMEM((_CHUNK, _D), jnp.float32),
        pltpu.SemaphoreType.DMA,
    ],
)
def _gather_rows(idx_hbm, table_hbm, out_hbm, idx_v, rows_v, gsem):
    wid = lax.axis_index("s") * _NC + lax.axis_index("c")
    rbase = pl.multiple_of(wid * _RPW, _RPW)
    pltpu.sync_copy(idx_hbm.at[pl.ds(rbase, _RPW)], idx_v)

    def body(j, carry):
        off = pl.multiple_of(j * _CHUNK, _CHUNK)
        idx_slice = idx_v.at[pl.ds(off, _CHUNK)]
        pltpu.async_copy(table_hbm.at[idx_slice], rows_v, gsem).wait()
        pltpu.sync_copy(rows_v, out_hbm.at[pl.ds(rbase + off, _CHUNK)])
        return carry

    lax.fori_loop(0, _CPW, body, 0)


def kernel(indices, table):
    idx_flat = indices.astype(jnp.int32).reshape(_R)
    out = _gather_rows(idx_flat, table)
    return out.reshape(_B, _F * _D)


# speedup vs baseline: 1.7825x; 1.1252x over previous
import functools

import jax
import jax.numpy as jnp
from jax import lax
from jax.experimental import pallas as pl
from jax.experimental.pallas import tpu as pltpu
from jax.experimental.pallas import tpu_sc as plsc

_B = 4096
_F = 26
_D = 64
_R = _B * _F
_NC = 2
_NS = 16
_NW = _NC * _NS
_CHUNK = 128
_RPW = _R // _NW
_CPW = _RPW // _CHUNK

_mesh = plsc.VectorSubcoreMesh(core_axis_name="c", subcore_axis_name="s")


@functools.partial(
    pl.kernel,
    mesh=_mesh,
    compiler_params=pltpu.CompilerParams(use_tc_tiling_on_sc=False),
    out_type=jax.ShapeDtypeStruct((_R, _D), jnp.float32),
    scratch_types=[
        pltpu.VMEM((_RPW,), jnp.int32),
        pltpu.VMEM((4, _CHUNK, _D), jnp.float32),
        pltpu.SemaphoreType.DMA,
        pltpu.SemaphoreType.DMA,
    ],
)
def _gather_rows(idx_hbm, table_hbm, out_hbm, idx_v, rows_v, gsem, wsem):
    wid = lax.axis_index("s") * _NC + lax.axis_index("c")
    rbase = pl.multiple_of(wid * _RPW, _RPW)
    pltpu.sync_copy(idx_hbm.at[pl.ds(rbase, _RPW)], idx_v)

    def g_desc(j, p):
        off = j * _CHUNK if isinstance(j, int) else pl.multiple_of(j * _CHUNK, _CHUNK)
        return pltpu.make_async_copy(
            table_hbm.at[idx_v.at[pl.ds(off, _CHUNK)]], rows_v.at[p], gsem)

    def w_desc(j, p):
        off = j * _CHUNK if isinstance(j, int) else pl.multiple_of(j * _CHUNK, _CHUNK)
        return pltpu.make_async_copy(
            rows_v.at[p], out_hbm.at[pl.ds(rbase + off, _CHUNK)], wsem)

    g_desc(0, 0).start()
    g_desc(1, 1).start()
    for j in (0, 1):
        g_desc(j, j).wait()
        w_desc(j, j).start()
        g_desc(j + 2, j + 2).start()

    def body(i, carry):
        for k in range(4):
            j = i * 4 + 2 + k
            p = (2 + k) % 4
            g_desc(j, p).wait()
            w_desc(j, p).start()
            w_desc(j - 2, (p + 2) % 4).wait()
            g_desc(j + 2, (p + 2) % 4).start()
        return carry

    lax.fori_loop(0, (_CPW - 6) // 4, body, 0)

    for j in (22, 23):
        g_desc(j, j % 4).wait()
        w_desc(j, j % 4).start()
        w_desc(j - 2, (j - 2) % 4).wait()
        g_desc(j + 2, (j + 2) % 4).start()
    for j in (24, 25):
        g_desc(j, j % 4).wait()
        w_desc(j, j % 4).start()
        w_desc(j - 2, (j - 2) % 4).wait()
    w_desc(24, 0).wait()
    w_desc(25, 1).wait()


def kernel(indices, table):
    idx_flat = indices.astype(jnp.int32).reshape(_R)
    out = _gather_rows(idx_flat, table)
    return out.reshape(_B, _F * _D)


# speedup vs baseline: 1.8051x; 1.0127x over previous
import functools

import jax
import jax.numpy as jnp
from jax import lax
from jax.experimental import pallas as pl
from jax.experimental.pallas import tpu as pltpu
from jax.experimental.pallas import tpu_sc as plsc

_B = 4096
_F = 26
_D = 64
_R = _B * _F
_NC = 2
_NS = 16
_NW = _NC * _NS
_CHUNK = 416
_RPW = _R // _NW
_CPW = _RPW // _CHUNK
_NBUF = 4

_mesh = plsc.VectorSubcoreMesh(core_axis_name="c", subcore_axis_name="s")


@functools.partial(
    pl.kernel,
    mesh=_mesh,
    compiler_params=pltpu.CompilerParams(use_tc_tiling_on_sc=False),
    out_type=jax.ShapeDtypeStruct((_R, _D), jnp.float32),
    scratch_types=[
        pltpu.VMEM((_RPW,), jnp.int32),
        pltpu.VMEM((_NBUF, _CHUNK, _D), jnp.float32),
        pltpu.SemaphoreType.DMA,
        pltpu.SemaphoreType.DMA,
    ],
)
def _gather_rows(idx_hbm, table_hbm, out_hbm, idx_v, rows_v, gsem, wsem):
    wid = lax.axis_index("s") * _NC + lax.axis_index("c")
    rbase = pl.multiple_of(wid * _RPW, _RPW)
    pltpu.sync_copy(idx_hbm.at[pl.ds(rbase, _RPW)], idx_v)

    def g_desc(j):
        return pltpu.make_async_copy(
            table_hbm.at[idx_v.at[pl.ds(j * _CHUNK, _CHUNK)]],
            rows_v.at[j % _NBUF], gsem)

    def w_desc(j):
        return pltpu.make_async_copy(
            rows_v.at[j % _NBUF],
            out_hbm.at[pl.ds(rbase + j * _CHUNK, _CHUNK)], wsem)

    g_desc(0).start()
    g_desc(1).start()
    for j in range(_CPW):
        g_desc(j).wait()
        w_desc(j).start()
        if j >= 2:
            w_desc(j - 2).wait()
        if j + 2 < _CPW:
            g_desc(j + 2).start()
    w_desc(_CPW - 2).wait()
    w_desc(_CPW - 1).wait()


def kernel(indices, table):
    idx_flat = indices.astype(jnp.int32).reshape(_R)
    out = _gather_rows(idx_flat, table)
    return out.reshape(_B, _F * _D)
